# baseline (speedup 1.0000x reference)
import functools

import jax
import jax.numpy as jnp
from jax import lax
from jax.experimental import pallas as pl
from jax.experimental.pallas import tpu as pltpu
from jax.experimental.pallas import tpu_sc as plsc

F32 = jnp.float32

_NC = 2
_NS = 16
_NW = _NC * _NS
_CH = 40
_NBUF = 10


@functools.lru_cache(maxsize=None)
def _make_agg(n, e, w):
    ew = e // _NW
    nch = ew // _CH
    rpt = n // _NS
    mesh = plsc.VectorSubcoreMesh(
        core_axis_name="c", subcore_axis_name="s", num_cores=_NC,
        num_subcores=_NS)

    ngrp = nch // _NBUF

    @functools.partial(
        pl.kernel,
        out_type=jax.ShapeDtypeStruct((_NC, n, w), F32),
        mesh=mesh,
        compiler_params=pltpu.CompilerParams(use_tc_tiling_on_sc=False),
        scratch_types=[
            pltpu.VMEM((nch, _CH), jnp.int32),
            pltpu.VMEM((nch, _CH), jnp.int32),
            pltpu.VMEM((_NBUF, _CH, w), F32),
            pltpu.VMEM_SHARED((n, w), F32),
        ] + [pltpu.SemaphoreType.DMA] * (2 * _NBUF),
    )
    def agg(table, ei3, zeros, out, srcv, dstv, rows, acc, *sems):
        gsem = sems[:_NBUF]
        ssem = sems[_NBUF:]
        c = lax.axis_index("c")
        s = lax.axis_index("s")
        wid = s * _NC + c
        pltpu.sync_copy(zeros.at[pl.ds(s * rpt, rpt)],
                        acc.at[pl.ds(s * rpt, rpt)])
        pltpu.sync_copy(ei3.at[0, pl.ds(wid * nch, nch)], srcv)
        pltpu.sync_copy(ei3.at[1, pl.ds(wid * nch, nch)], dstv)
        for b in range(_NBUF):
            pltpu.async_copy(table.at[srcv.at[b]], rows.at[b], gsem[b])
        plsc.subcore_barrier()

        def group(g, carry):
            g0 = g * _NBUF
            for b in range(_NBUF):
                j = g0 + b
                pltpu.make_async_copy(table.at[srcv.at[j]], rows.at[b],
                                      gsem[b]).wait()
                pltpu.async_copy(rows.at[b], acc.at[dstv.at[j]], ssem[b],
                                 add=True)
            for b in range(_NBUF):
                j = g0 + b

                @pl.when(g < ngrp - 1)
                def _():
                    pltpu.make_async_copy(rows.at[b], acc.at[dstv.at[j]],
                                          ssem[b]).wait()
                    pltpu.async_copy(table.at[srcv.at[j + _NBUF]],
                                     rows.at[b], gsem[b])
            return carry

        lax.fori_loop(0, ngrp, group, 0)
        for b in range(_NBUF):
            j = (ngrp - 1) * _NBUF + b
            pltpu.make_async_copy(rows.at[b], acc.at[dstv.at[j]],
                                  ssem[b]).wait()
        plsc.subcore_barrier()
        pltpu.sync_copy(acc.at[pl.ds(s * rpt, rpt)],
                        out.at[c, pl.ds(s * rpt, rpt)])

    return agg


def _pre_body(x_ref, wl_ref, t_ref):
    t = jnp.dot(x_ref[...], wl_ref[...], preferred_element_type=F32)
    col = lax.broadcasted_iota(jnp.int32, t.shape, 1)
    h = wl_ref.shape[1] - 16
    t_ref[0:t.shape[0], :] = jnp.where(col == h, 1.0, t)


def _bn(v, g, b):
    mu = jnp.mean(v, axis=0, keepdims=True)
    var = jnp.mean((v - mu) ** 2, axis=0, keepdims=True)
    return (v - mu) / jnp.sqrt(var + 1e-5) * g + b


def _post_common(p_ref, cnt, hprev_ref, wr_ref, bl_ref, g_ref, b_ref):
    h = bl_ref.shape[1]
    n = hprev_ref.shape[0]
    ssum = p_ref[0, 0:n, 0:h] + p_ref[1, 0:n, 0:h]
    mean = ssum / jnp.maximum(cnt, 1.0)
    pre = mean + bl_ref[...] + jnp.dot(
        hprev_ref[...], wr_ref[...], preferred_element_type=F32)
    return jax.nn.relu(_bn(pre, g_ref[...], b_ref[...]))


def _post0_body(p_ref, hprev_ref, wr_ref, bl_ref, g_ref, b_ref, wln_ref,
                h_ref, t_ref, cnt_ref):
    n = hprev_ref.shape[0]
    cnt = p_ref[0, 0:n, 64:65] + p_ref[1, 0:n, 64:65]
    hnew = _post_common(p_ref, cnt, hprev_ref, wr_ref, bl_ref, g_ref, b_ref)
    h_ref[...] = hnew
    cnt_ref[...] = cnt
    t_ref[0:hnew.shape[0], :] = jnp.dot(
        hnew, wln_ref[...], preferred_element_type=F32)


def _post1_body(p_ref, cnt_ref, hprev_ref, wr_ref, bl_ref, g_ref, b_ref,
                wln_ref, h_ref, t_ref):
    hnew = _post_common(p_ref, cnt_ref[...], hprev_ref, wr_ref, bl_ref,
                        g_ref, b_ref)
    h_ref[...] = hnew
    t_ref[0:hnew.shape[0], :] = jnp.dot(
        hnew, wln_ref[...], preferred_element_type=F32)


def _final_body(p_ref, cnt_ref, hprev_ref, wr_ref, bl_ref, g_ref, b_ref,
                batch_ref, rad_ref, radg_ref, radb_ref,
                cw1a_ref, cw1b_ref, cb1_ref, cw2_ref, cb2_ref,
                cw3_ref, cb3_ref, ewa_ref, ewb_ref, eb_ref,
                logits_ref, emb_ref, node_ref):
    h3 = _post_common(p_ref, cnt_ref[...], hprev_ref, wr_ref, bl_ref,
                      g_ref, b_ref)
    node_ref[...] = h3
    nb = ewa_ref.shape[0]
    n = h3.shape[0]
    gid = lax.broadcasted_iota(jnp.int32, (nb, n), 0)
    onehot = jnp.where(gid == batch_ref[...], 1.0, 0.0)
    pooled = jnp.dot(onehot, h3, preferred_element_type=F32)
    cntb = jnp.sum(onehot, axis=1, keepdims=True)
    gemb = pooled / jnp.maximum(cntb, 1.0)
    rbn = _bn(rad_ref[...], radg_ref[...], radb_ref[...])
    z = jax.nn.relu(
        jnp.dot(gemb, cw1a_ref[...], preferred_element_type=F32)
        + jnp.dot(rbn, cw1b_ref[...], preferred_element_type=F32,
                  )
        + cb1_ref[...])
    z = jax.nn.relu(jnp.dot(z, cw2_ref[...], preferred_element_type=F32,
                            ) + cb2_ref[...])
    logits_ref[...] = jnp.dot(z, cw3_ref[...], preferred_element_type=F32,
                              ) + cb3_ref[...]
    emb_ref[...] = (
        jnp.dot(gemb, ewa_ref[...], preferred_element_type=F32)
        + jnp.dot(rbn, ewb_ref[...], preferred_element_type=F32,
                  )
        + eb_ref[...])


def _row(v):
    return v.reshape(1, -1)


def kernel(x, edge_index, batch, radiomics, Wl0, bl0, Wr0, bn0_g, bn0_b,
           Wl1, bl1, Wr1, bn1_g, bn1_b, Wl2, bl2, Wr2, bn2_g, bn2_b,
           rad_g, rad_b, cW1, cb1, cW2, cb2, cW3, cb3, eW, eb):
    n, din = x.shape
    e = edge_index.shape[1]
    h = Wl0.shape[1]
    nb, rad = radiomics.shape
    w0 = h + 16
    npad = -(-n // 128) * 128

    ei3 = edge_index.reshape(2, e // _CH, _CH)

    wl0p = jnp.concatenate([Wl0, jnp.zeros((din, 16), F32)], axis=1)
    t0 = pl.pallas_call(
        _pre_body,
        out_shape=jax.ShapeDtypeStruct((npad, w0), F32),
    )(x, wl0p)

    z80 = jnp.zeros((npad, w0), F32)
    z64 = jnp.zeros((npad, h), F32)

    p0 = _make_agg(npad, e, w0)(t0, ei3, z80)

    h1, t1, cnt = pl.pallas_call(
        _post0_body,
        out_shape=(
            jax.ShapeDtypeStruct((n, h), F32),
            jax.ShapeDtypeStruct((npad, h), F32),
            jax.ShapeDtypeStruct((n, 1), F32),
        ),
    )(p0, x, Wr0, _row(bl0), _row(bn0_g), _row(bn0_b), Wl1)

    p1 = _make_agg(npad, e, h)(t1, ei3, z64)

    h2, t2 = pl.pallas_call(
        _post1_body,
        out_shape=(
            jax.ShapeDtypeStruct((n, h), F32),
            jax.ShapeDtypeStruct((npad, h), F32),
        ),
    )(p1, cnt, h1, Wr1, _row(bl1), _row(bn1_g), _row(bn1_b), Wl2)

    p2 = _make_agg(npad, e, h)(t2, ei3, z64)

    logits, emb, node_emb = pl.pallas_call(
        _final_body,
        out_shape=(
            jax.ShapeDtypeStruct((nb, 2), F32),
            jax.ShapeDtypeStruct((nb, h + rad), F32),
            jax.ShapeDtypeStruct((n, h), F32),
        ),
    )(p2, cnt, h2, Wr2, _row(bl2), _row(bn2_g), _row(bn2_b),
      _row(batch), radiomics, _row(rad_g), _row(rad_b),
      cW1[:h], cW1[h:], _row(cb1), cW2, _row(cb2), cW3, _row(cb3),
      eW[:h], eW[h:], _row(eb))

    return logits, emb, node_emb


# speedup vs baseline: 1.0172x; 1.0172x over previous
import functools

import jax
import jax.numpy as jnp
from jax import lax
from jax.experimental import pallas as pl
from jax.experimental.pallas import tpu as pltpu
from jax.experimental.pallas import tpu_sc as plsc

F32 = jnp.float32

_NC = 2
_NS = 16
_NW = _NC * _NS
_CH = 80
_NBUF = 5


@functools.lru_cache(maxsize=None)
def _make_agg(n, e, w):
    ew = e // _NW
    nch = ew // _CH
    rpt = n // _NS
    mesh = plsc.VectorSubcoreMesh(
        core_axis_name="c", subcore_axis_name="s", num_cores=_NC,
        num_subcores=_NS)

    ngrp = nch // _NBUF

    @functools.partial(
        pl.kernel,
        out_type=jax.ShapeDtypeStruct((_NC, n, w), F32),
        mesh=mesh,
        compiler_params=pltpu.CompilerParams(use_tc_tiling_on_sc=False),
        scratch_types=[
            pltpu.VMEM((nch, _CH), jnp.int32),
            pltpu.VMEM((nch, _CH), jnp.int32),
            pltpu.VMEM((_NBUF, _CH, w), F32),
            pltpu.VMEM_SHARED((n, w), F32),
        ] + [pltpu.SemaphoreType.DMA] * (2 * _NBUF),
    )
    def agg(table, ei3, zeros, out, srcv, dstv, rows, acc, *sems):
        gsem = sems[:_NBUF]
        ssem = sems[_NBUF:]
        c = lax.axis_index("c")
        s = lax.axis_index("s")
        wid = s * _NC + c
        pltpu.sync_copy(zeros.at[pl.ds(s * rpt, rpt)],
                        acc.at[pl.ds(s * rpt, rpt)])
        pltpu.sync_copy(ei3.at[0, pl.ds(wid * nch, nch)], srcv)
        pltpu.sync_copy(ei3.at[1, pl.ds(wid * nch, nch)], dstv)
        for b in range(_NBUF):
            pltpu.async_copy(table.at[srcv.at[b]], rows.at[b], gsem[b])
        plsc.subcore_barrier()

        def group(g, carry):
            g0 = g * _NBUF
            for b in range(_NBUF):
                j = g0 + b
                pltpu.make_async_copy(table.at[srcv.at[j]], rows.at[b],
                                      gsem[b]).wait()
                pltpu.async_copy(rows.at[b], acc.at[dstv.at[j]], ssem[b],
                                 add=True)
            for b in range(_NBUF):
                j = g0 + b

                @pl.when(g < ngrp - 1)
                def _():
                    pltpu.make_async_copy(rows.at[b], acc.at[dstv.at[j]],
                                          ssem[b]).wait()
                    pltpu.async_copy(table.at[srcv.at[j + _NBUF]],
                                     rows.at[b], gsem[b])
            return carry

        lax.fori_loop(0, ngrp, group, 0)
        for b in range(_NBUF):
            j = (ngrp - 1) * _NBUF + b
            pltpu.make_async_copy(rows.at[b], acc.at[dstv.at[j]],
                                  ssem[b]).wait()
        plsc.subcore_barrier()
        pltpu.sync_copy(acc.at[pl.ds(s * rpt, rpt)],
                        out.at[c, pl.ds(s * rpt, rpt)])

    return agg


def _pre_body(x_ref, wl_ref, t_ref):
    t = jnp.dot(x_ref[...], wl_ref[...], preferred_element_type=F32)
    col = lax.broadcasted_iota(jnp.int32, t.shape, 1)
    h = wl_ref.shape[1] - 16
    t_ref[0:t.shape[0], :] = jnp.where(col == h, 1.0, t)


def _rmat_body(h_ref, wr_ref, bl_ref, r_ref):
    r_ref[...] = jnp.dot(h_ref[...], wr_ref[...],
                         preferred_element_type=F32) + bl_ref[...]


def _bn(v, g, b):
    mu = jnp.mean(v, axis=0, keepdims=True)
    var = jnp.mean((v - mu) ** 2, axis=0, keepdims=True)
    return (v - mu) / jnp.sqrt(var + 1e-5) * g + b


def _post_common(p_ref, cnt, r_ref, g_ref, b_ref):
    h = g_ref.shape[1]
    n = r_ref.shape[0]
    ssum = p_ref[0, 0:n, 0:h] + p_ref[1, 0:n, 0:h]
    mean = ssum / jnp.maximum(cnt, 1.0)
    pre = mean + r_ref[...]
    return jax.nn.relu(_bn(pre, g_ref[...], b_ref[...]))


def _post0_body(p_ref, r_ref, g_ref, b_ref, wln_ref, h_ref, t_ref, cnt_ref):
    n = r_ref.shape[0]
    cnt = p_ref[0, 0:n, 64:65] + p_ref[1, 0:n, 64:65]
    hnew = _post_common(p_ref, cnt, r_ref, g_ref, b_ref)
    h_ref[...] = hnew
    cnt_ref[...] = cnt
    t_ref[0:n, :] = jnp.dot(hnew, wln_ref[...], preferred_element_type=F32)


def _post1_body(p_ref, cnt_ref, r_ref, g_ref, b_ref, wln_ref, h_ref, t_ref):
    hnew = _post_common(p_ref, cnt_ref[...], r_ref, g_ref, b_ref)
    h_ref[...] = hnew
    t_ref[0:r_ref.shape[0], :] = jnp.dot(hnew, wln_ref[...],
                                         preferred_element_type=F32)


def _final_body(p_ref, cnt_ref, r_ref, g_ref, b_ref,
                batch_ref, rad_ref, radg_ref, radb_ref,
                cw1a_ref, cw1b_ref, cb1_ref, cw2_ref, cb2_ref,
                cw3_ref, cb3_ref, ewa_ref, ewb_ref, eb_ref,
                logits_ref, emb_ref, node_ref):
    h3 = _post_common(p_ref, cnt_ref[...], r_ref, g_ref, b_ref)
    node_ref[...] = h3
    nb = ewa_ref.shape[0]
    n = h3.shape[0]
    gid = lax.broadcasted_iota(jnp.int32, (nb, n), 0)
    onehot = jnp.where(gid == batch_ref[...], 1.0, 0.0)
    pooled = jnp.dot(onehot, h3, preferred_element_type=F32)
    cntb = jnp.sum(onehot, axis=1, keepdims=True)
    gemb = pooled / jnp.maximum(cntb, 1.0)
    rbn = _bn(rad_ref[...], radg_ref[...], radb_ref[...])
    z = jax.nn.relu(
        jnp.dot(gemb, cw1a_ref[...], preferred_element_type=F32)
        + jnp.dot(rbn, cw1b_ref[...], preferred_element_type=F32)
        + cb1_ref[...])
    z = jax.nn.relu(jnp.dot(z, cw2_ref[...], preferred_element_type=F32)
                    + cb2_ref[...])
    logits_ref[...] = jnp.dot(z, cw3_ref[...],
                              preferred_element_type=F32) + cb3_ref[...]
    emb_ref[...] = (
        jnp.dot(gemb, ewa_ref[...], preferred_element_type=F32)
        + jnp.dot(rbn, ewb_ref[...], preferred_element_type=F32)
        + eb_ref[...])


def _row(v):
    return v.reshape(1, -1)


def kernel(x, edge_index, batch, radiomics, Wl0, bl0, Wr0, bn0_g, bn0_b,
           Wl1, bl1, Wr1, bn1_g, bn1_b, Wl2, bl2, Wr2, bn2_g, bn2_b,
           rad_g, rad_b, cW1, cb1, cW2, cb2, cW3, cb3, eW, eb):
    n, din = x.shape
    e = edge_index.shape[1]
    h = Wl0.shape[1]
    nb, rad = radiomics.shape
    w0 = h + 16
    npad = -(-n // 128) * 128

    ei3 = edge_index.reshape(2, e // _CH, _CH)

    wl0p = jnp.concatenate([Wl0, jnp.zeros((din, 16), F32)], axis=1)
    t0 = pl.pallas_call(
        _pre_body,
        out_shape=jax.ShapeDtypeStruct((npad, w0), F32),
    )(x, wl0p)

    z80 = jnp.zeros((npad, w0), F32)
    z64 = jnp.zeros((npad, h), F32)

    def rmat(hin, wr, bl):
        return pl.pallas_call(
            _rmat_body,
            out_shape=jax.ShapeDtypeStruct((n, h), F32),
        )(hin, wr, _row(bl))

    p0 = _make_agg(npad, e, w0)(t0, ei3, z80)
    r0 = rmat(x, Wr0, bl0)

    h1, t1, cnt = pl.pallas_call(
        _post0_body,
        out_shape=(
            jax.ShapeDtypeStruct((n, h), F32),
            jax.ShapeDtypeStruct((npad, h), F32),
            jax.ShapeDtypeStruct((n, 1), F32),
        ),
    )(p0, r0, _row(bn0_g), _row(bn0_b), Wl1)

    p1 = _make_agg(npad, e, h)(t1, ei3, z64)
    r1 = rmat(h1, Wr1, bl1)

    h2, t2 = pl.pallas_call(
        _post1_body,
        out_shape=(
            jax.ShapeDtypeStruct((n, h), F32),
            jax.ShapeDtypeStruct((npad, h), F32),
        ),
    )(p1, cnt, r1, _row(bn1_g), _row(bn1_b), Wl2)

    p2 = _make_agg(npad, e, h)(t2, ei3, z64)
    r2 = rmat(h2, Wr2, bl2)

    logits, emb, node_emb = pl.pallas_call(
        _final_body,
        out_shape=(
            jax.ShapeDtypeStruct((nb, 2), F32),
            jax.ShapeDtypeStruct((nb, h + rad), F32),
            jax.ShapeDtypeStruct((n, h), F32),
        ),
    )(p2, cnt, r2, _row(bn2_g), _row(bn2_b),
      _row(batch), radiomics, _row(rad_g), _row(rad_b),
      cW1[:h], cW1[h:], _row(cb1), cW2, _row(cb2), cW3, _row(cb3),
      eW[:h], eW[h:], _row(eb))

    return logits, emb, node_emb


# speedup vs baseline: 1.0208x; 1.0035x over previous
import functools

import jax
import jax.numpy as jnp
from jax import lax
from jax.experimental import pallas as pl
from jax.experimental.pallas import tpu as pltpu
from jax.experimental.pallas import tpu_sc as plsc

F32 = jnp.float32

_NC = 2
_NS = 16
_NW = _NC * _NS
_CH = 80
_NBUF = 5


@functools.lru_cache(maxsize=None)
def _make_agg(n, e, w):
    ew = e // _NW
    nch = ew // _CH
    rpt = n // _NS
    mesh = plsc.VectorSubcoreMesh(
        core_axis_name="c", subcore_axis_name="s", num_cores=_NC,
        num_subcores=_NS)

    ngrp = nch // _NBUF

    @functools.partial(
        pl.kernel,
        out_type=jax.ShapeDtypeStruct((_NC, n, w), F32),
        mesh=mesh,
        compiler_params=pltpu.CompilerParams(use_tc_tiling_on_sc=False),
        scratch_types=[
            pltpu.VMEM((nch, _CH), jnp.int32),
            pltpu.VMEM((nch, _CH), jnp.int32),
            pltpu.VMEM((_NBUF, _CH, w), F32),
            pltpu.VMEM_SHARED((n, w), F32),
        ] + [pltpu.SemaphoreType.DMA] * (2 * _NBUF),
    )
    def agg(table, ei3, zeros, out, srcv, dstv, rows, acc, *sems):
        gsem = sems[:_NBUF]
        ssem = sems[_NBUF:]
        c = lax.axis_index("c")
        s = lax.axis_index("s")
        wid = s * _NC + c
        pltpu.sync_copy(zeros.at[pl.ds(s * rpt, rpt)],
                        acc.at[pl.ds(s * rpt, rpt)])
        pltpu.sync_copy(ei3.at[0, pl.ds(wid * nch, nch)], srcv)
        pltpu.sync_copy(ei3.at[1, pl.ds(wid * nch, nch)], dstv)
        for b in range(_NBUF):
            pltpu.async_copy(table.at[srcv.at[b]], rows.at[b], gsem[b])
        plsc.subcore_barrier()

        def group(g, carry):
            g0 = g * _NBUF
            for b in range(_NBUF):
                j = g0 + b
                pltpu.make_async_copy(table.at[srcv.at[j]], rows.at[b],
                                      gsem[b]).wait()
                pltpu.async_copy(rows.at[b], acc.at[dstv.at[j]], ssem[b],
                                 add=True)
            for b in range(_NBUF):
                j = g0 + b

                @pl.when(g < ngrp - 1)
                def _():
                    pltpu.make_async_copy(rows.at[b], acc.at[dstv.at[j]],
                                          ssem[b]).wait()
                    pltpu.async_copy(table.at[srcv.at[j + _NBUF]],
                                     rows.at[b], gsem[b])
            return carry

        lax.fori_loop(0, ngrp, group, 0)
        for b in range(_NBUF):
            j = (ngrp - 1) * _NBUF + b
            pltpu.make_async_copy(rows.at[b], acc.at[dstv.at[j]],
                                  ssem[b]).wait()
        plsc.subcore_barrier()
        pltpu.sync_copy(acc.at[pl.ds(s * rpt, rpt)],
                        out.at[c, pl.ds(s * rpt, rpt)])

    return agg


def _pre_body(x_ref, wl_ref, t_ref):
    t = jnp.dot(x_ref[...], wl_ref[...], preferred_element_type=F32)
    col = lax.broadcasted_iota(jnp.int32, t.shape, 1)
    h = wl_ref.shape[1] - 16
    t_ref[0:t.shape[0], :] = jnp.where(col == h, 1.0, t)


def _rmat_body(h_ref, wr_ref, bl_ref, r_ref):
    r_ref[...] = jnp.dot(h_ref[...], wr_ref[...],
                         preferred_element_type=F32) + bl_ref[...]


def _bn(v, g, b):
    mu = jnp.mean(v, axis=0, keepdims=True)
    var = jnp.mean((v - mu) ** 2, axis=0, keepdims=True)
    return (v - mu) / jnp.sqrt(var + 1e-5) * g + b


def _post_common(p_ref, cnt, r_ref, g_ref, b_ref):
    h = g_ref.shape[1]
    n = r_ref.shape[0]
    ssum = p_ref[0, 0:n, 0:h] + p_ref[1, 0:n, 0:h]
    mean = ssum / jnp.maximum(cnt, 1.0)
    pre = mean + r_ref[...]
    return jax.nn.relu(_bn(pre, g_ref[...], b_ref[...]))


def _post0_body(p_ref, r_ref, g_ref, b_ref, wln_ref, wrn_ref, bln_ref,
                t_ref, rn_ref, cnt_ref):
    n = r_ref.shape[0]
    cnt = p_ref[0, 0:n, 64:65] + p_ref[1, 0:n, 64:65]
    hnew = _post_common(p_ref, cnt, r_ref, g_ref, b_ref)
    cnt_ref[...] = cnt
    t_ref[0:n, :] = jnp.dot(hnew, wln_ref[...], preferred_element_type=F32)
    rn_ref[...] = jnp.dot(hnew, wrn_ref[...],
                          preferred_element_type=F32) + bln_ref[...]


def _post1_body(p_ref, cnt_ref, r_ref, g_ref, b_ref, wln_ref, wrn_ref,
                bln_ref, t_ref, rn_ref):
    hnew = _post_common(p_ref, cnt_ref[...], r_ref, g_ref, b_ref)
    t_ref[0:r_ref.shape[0], :] = jnp.dot(hnew, wln_ref[...],
                                         preferred_element_type=F32)
    rn_ref[...] = jnp.dot(hnew, wrn_ref[...],
                          preferred_element_type=F32) + bln_ref[...]


def _final_body(p_ref, cnt_ref, r_ref, g_ref, b_ref,
                batch_ref, rad_ref, radg_ref, radb_ref,
                cw1a_ref, cw1b_ref, cb1_ref, cw2_ref, cb2_ref,
                cw3_ref, cb3_ref, ewa_ref, ewb_ref, eb_ref,
                logits_ref, emb_ref, node_ref):
    h3 = _post_common(p_ref, cnt_ref[...], r_ref, g_ref, b_ref)
    node_ref[...] = h3
    nb = ewa_ref.shape[0]
    n = h3.shape[0]
    gid = lax.broadcasted_iota(jnp.int32, (nb, n), 0)
    onehot = jnp.where(gid == batch_ref[...], 1.0, 0.0)
    pooled = jnp.dot(onehot, h3, preferred_element_type=F32)
    cntb = jnp.sum(onehot, axis=1, keepdims=True)
    gemb = pooled / jnp.maximum(cntb, 1.0)
    rbn = _bn(rad_ref[...], radg_ref[...], radb_ref[...])
    z = jax.nn.relu(
        jnp.dot(gemb, cw1a_ref[...], preferred_element_type=F32)
        + jnp.dot(rbn, cw1b_ref[...], preferred_element_type=F32)
        + cb1_ref[...])
    z = jax.nn.relu(jnp.dot(z, cw2_ref[...], preferred_element_type=F32)
                    + cb2_ref[...])
    logits_ref[...] = jnp.dot(z, cw3_ref[...],
                              preferred_element_type=F32) + cb3_ref[...]
    emb_ref[...] = (
        jnp.dot(gemb, ewa_ref[...], preferred_element_type=F32)
        + jnp.dot(rbn, ewb_ref[...], preferred_element_type=F32)
        + eb_ref[...])


def _row(v):
    return v.reshape(1, -1)


def kernel(x, edge_index, batch, radiomics, Wl0, bl0, Wr0, bn0_g, bn0_b,
           Wl1, bl1, Wr1, bn1_g, bn1_b, Wl2, bl2, Wr2, bn2_g, bn2_b,
           rad_g, rad_b, cW1, cb1, cW2, cb2, cW3, cb3, eW, eb):
    n, din = x.shape
    e = edge_index.shape[1]
    h = Wl0.shape[1]
    nb, rad = radiomics.shape
    w0 = h + 16
    npad = -(-n // 128) * 128

    ei3 = edge_index.reshape(2, e // _CH, _CH)

    wl0p = jnp.concatenate([Wl0, jnp.zeros((din, 16), F32)], axis=1)
    t0 = pl.pallas_call(
        _pre_body,
        out_shape=jax.ShapeDtypeStruct((npad, w0), F32),
    )(x, wl0p)

    z80 = jnp.zeros((npad, w0), F32)
    z64 = jnp.zeros((npad, h), F32)

    def rmat(hin, wr, bl):
        return pl.pallas_call(
            _rmat_body,
            out_shape=jax.ShapeDtypeStruct((n, h), F32),
        )(hin, wr, _row(bl))

    p0 = _make_agg(npad, e, w0)(t0, ei3, z80)
    r0 = rmat(x, Wr0, bl0)

    t1, r1, cnt = pl.pallas_call(
        _post0_body,
        out_shape=(
            jax.ShapeDtypeStruct((npad, h), F32),
            jax.ShapeDtypeStruct((n, h), F32),
            jax.ShapeDtypeStruct((n, 1), F32),
        ),
    )(p0, r0, _row(bn0_g), _row(bn0_b), Wl1, Wr1, _row(bl1))

    p1 = _make_agg(npad, e, h)(t1, ei3, z64)

    t2, r2 = pl.pallas_call(
        _post1_body,
        out_shape=(
            jax.ShapeDtypeStruct((npad, h), F32),
            jax.ShapeDtypeStruct((n, h), F32),
        ),
    )(p1, cnt, r1, _row(bn1_g), _row(bn1_b), Wl2, Wr2, _row(bl2))

    p2 = _make_agg(npad, e, h)(t2, ei3, z64)

    logits, emb, node_emb = pl.pallas_call(
        _final_body,
        out_shape=(
            jax.ShapeDtypeStruct((nb, 2), F32),
            jax.ShapeDtypeStruct((nb, h + rad), F32),
            jax.ShapeDtypeStruct((n, h), F32),
        ),
    )(p2, cnt, r2, _row(bn2_g), _row(bn2_b),
      _row(batch), radiomics, _row(rad_g), _row(rad_b),
      cW1[:h], cW1[h:], _row(cb1), cW2, _row(cb2), cW3, _row(cb3),
      eW[:h], eW[h:], _row(eb))

    return logits, emb, node_emb
